# initial kernel scaffold (unmeasured)
import jax
import jax.numpy as jnp
from jax import lax
from jax.experimental import pallas as pl
from jax.experimental.pallas import tpu as pltpu

N_SLABS = 8
N_STEPS = 4


def kernel(dy, W):
    m, k_shard = dy.shape
    d = W.shape[0]
    slab = k_shard // N_SLABS

    def body(dy_ref, w_ref, out_ref, acc_ref, recv_ref, send_sems, recv_sems):
        x = lax.axis_index("x")
        y = lax.axis_index("y")
        z = lax.axis_index("z")
        c = x * 4 + z

        a = dy_ref[:, pl.ds(c * slab, slab)]
        b = w_ref[:, pl.ds(c * slab, slab)]
        acc_ref[...] = lax.dot_general(
            a, b,
            dimension_numbers=(((1,), (1,)), ((), ())),
            preferred_element_type=jnp.float32,
        )

        partners = [
            (1 - x, y, z),
            (x, 1 - y, z),
            (x, y, jnp.bitwise_xor(z, 1)),
            (x, y, jnp.bitwise_xor(z, 2)),
        ]
        for step, pid in enumerate(partners):
            rdma = pltpu.make_async_remote_copy(
                src_ref=acc_ref,
                dst_ref=recv_ref.at[step],
                send_sem=send_sems.at[step],
                recv_sem=recv_sems.at[step],
                device_id=pid,
                device_id_type=pl.DeviceIdType.MESH,
            )
            rdma.start()
            rdma.wait()
            acc_ref[...] += recv_ref[step]

        out_ref[...] = acc_ref[...]

    return pl.pallas_call(
        body,
        out_shape=jax.ShapeDtypeStruct((m, d), jnp.float32),
        in_specs=[
            pl.BlockSpec(memory_space=pltpu.VMEM),
            pl.BlockSpec(memory_space=pltpu.VMEM),
        ],
        out_specs=pl.BlockSpec(memory_space=pltpu.VMEM),
        scratch_shapes=[
            pltpu.VMEM((m, d), jnp.float32),
            pltpu.VMEM((N_STEPS, m, d), jnp.float32),
            pltpu.SemaphoreType.DMA((N_STEPS,)),
            pltpu.SemaphoreType.DMA((N_STEPS,)),
        ],
        compiler_params=pltpu.CompilerParams(collective_id=0),
    )(dy, W)


# baseline (device time: 263125 ns/iter reference)
import jax
import jax.numpy as jnp
from jax import lax
from jax.experimental import pallas as pl
from jax.experimental.pallas import tpu as pltpu

N_SLABS = 8
N_STEPS = 4


def kernel(dy, W):
    m, k_shard = dy.shape
    d = W.shape[0]
    slab = k_shard // N_SLABS

    def body(dy_ref, w_ref, out_ref, acc_ref, recv_ref, send_sems, recv_sems):
        x = lax.axis_index("x")
        y = lax.axis_index("y")
        z = lax.axis_index("z")
        c = x * 4 + z

        a = dy_ref[:, pl.ds(c * slab, slab)]
        b = w_ref[:, pl.ds(c * slab, slab)]
        acc_ref[...] = lax.dot_general(
            a, b,
            dimension_numbers=(((1,), (1,)), ((), ())),
            preferred_element_type=jnp.float32,
        )

        partners = [
            (1 - x, y, z),
            (x, 1 - y, z),
            (x, y, jnp.bitwise_xor(z, 1)),
            (x, y, jnp.bitwise_xor(z, 2)),
        ]
        for step, pid in enumerate(partners):
            rdma = pltpu.make_async_remote_copy(
                src_ref=acc_ref,
                dst_ref=recv_ref.at[step],
                send_sem=send_sems.at[step],
                recv_sem=recv_sems.at[step],
                device_id=pid,
                device_id_type=pl.DeviceIdType.MESH,
            )
            rdma.start()
            rdma.wait()
            acc_ref[...] += recv_ref[step]

        out_ref[...] = acc_ref[...]

    return pl.pallas_call(
        body,
        out_shape=jax.ShapeDtypeStruct((m, d), jnp.float32),
        in_specs=[
            pl.BlockSpec(memory_space=pltpu.VMEM),
            pl.BlockSpec(memory_space=pltpu.VMEM),
        ],
        out_specs=pl.BlockSpec(memory_space=pltpu.VMEM),
        scratch_shapes=[
            pltpu.VMEM((m, d), jnp.float32),
            pltpu.VMEM((N_STEPS, m, d), jnp.float32),
            pltpu.SemaphoreType.DMA((N_STEPS,)),
            pltpu.SemaphoreType.DMA((N_STEPS,)),
        ],
        compiler_params=pltpu.CompilerParams(
            vmem_limit_bytes=100 * 1024 * 1024,
        ),
    )(dy, W)


# device time: 63518 ns/iter; 4.1425x vs baseline; 4.1425x over previous
import jax
import jax.numpy as jnp
from jax import lax
from jax.experimental import pallas as pl
from jax.experimental.pallas import tpu as pltpu

N_SLABS = 8
NS = 2
CHUNK = 64
RS_OFF = (0, 8, 12, 14)
N_STEPS = 4


def kernel(dy, W):
    m, k_shard = dy.shape
    d = W.shape[0]
    slab = k_shard // N_SLABS
    csz = d // NS

    def body(dy_hbm, w_hbm, out_ref, a_buf, b_buf, acc, sbuf, rbuf, agbuf,
             in_sems, send_sems, recv_sems):
        x = lax.axis_index("x")
        y = lax.axis_index("y")
        z = lax.axis_index("z")
        c = x * 4 + z

        cpa = pltpu.make_async_copy(
            dy_hbm.at[:, pl.ds(c * slab, slab)], a_buf, in_sems.at[0])
        cpb = pltpu.make_async_copy(
            w_hbm.at[:, pl.ds(c * slab, slab)], b_buf, in_sems.at[1])
        cpa.start()
        cpb.start()
        cpa.wait()
        cpb.wait()
        acc[...] = lax.dot_general(
            a_buf[...].astype(jnp.bfloat16),
            b_buf[...].astype(jnp.bfloat16),
            dimension_numbers=(((1,), (1,)), ((), ())),
            preferred_element_type=jnp.float32,
        )

        info = {
            8: ((1 - x, y, z), x),
            4: ((x, 1 - y, z), y),
            1: ((x, y, jnp.bitwise_xor(z, 1)), z & 1),
            2: ((x, y, jnp.bitwise_xor(z, 2)), z >> 1),
        }
        base = (8, 4, 1, 2)
        orders = [
            tuple(base[(s + j * (N_STEPS // NS)) % N_STEPS] for s in range(N_STEPS))
            for j in range(NS)
        ]

        lo = [jnp.int32(0)] * NS
        sz = 16
        for s in range(N_STEPS):
            half = sz // 2
            rows = half * CHUNK
            off = RS_OFF[s] * CHUNK
            started = []
            for j in range(NS):
                pid, mybit = info[orders[j][s]]
                keep_lo = lo[j] + mybit * half
                send_lo = lo[j] + (1 - mybit) * half
                cols = pl.ds(j * csz, csz)
                sbuf[pl.ds(off, rows), cols] = (
                    acc[pl.ds(send_lo * CHUNK, rows), cols].astype(jnp.bfloat16))
                rdma = pltpu.make_async_remote_copy(
                    src_ref=sbuf.at[pl.ds(off, rows), cols],
                    dst_ref=rbuf.at[pl.ds(off, rows), cols],
                    send_sem=send_sems.at[s, j],
                    recv_sem=recv_sems.at[s, j],
                    device_id=pid,
                    device_id_type=pl.DeviceIdType.MESH,
                )
                rdma.start()
                started.append((rdma, keep_lo))
            for j, (rdma, keep_lo) in enumerate(started):
                cols = pl.ds(j * csz, csz)
                rdma.wait()
                acc[pl.ds(keep_lo * CHUNK, rows), cols] += (
                    rbuf[pl.ds(off, rows), cols].astype(jnp.float32))
                lo[j] = keep_lo
            sz = half

        for j in range(NS):
            cols = pl.ds(j * csz, csz)
            agbuf[pl.ds(lo[j] * CHUNK, CHUNK), cols] = (
                acc[pl.ds(lo[j] * CHUNK, CHUNK), cols].astype(jnp.bfloat16))

        sz = 1
        for s in range(N_STEPS):
            rows = sz * CHUNK
            started = []
            for j in range(NS):
                pid, mybit = info[orders[j][N_STEPS - 1 - s]]
                cols = pl.ds(j * csz, csz)
                rdma = pltpu.make_async_remote_copy(
                    src_ref=agbuf.at[pl.ds(lo[j] * CHUNK, rows), cols],
                    dst_ref=agbuf.at[pl.ds(lo[j] * CHUNK, rows), cols],
                    send_sem=send_sems.at[N_STEPS + s, j],
                    recv_sem=recv_sems.at[N_STEPS + s, j],
                    device_id=pid,
                    device_id_type=pl.DeviceIdType.MESH,
                )
                rdma.start()
                started.append((rdma, mybit))
            for j, (rdma, mybit) in enumerate(started):
                rdma.wait()
                lo[j] = lo[j] - mybit * sz
            sz *= 2

        out_ref[...] = agbuf[...].astype(jnp.float32)

    return pl.pallas_call(
        body,
        out_shape=jax.ShapeDtypeStruct((m, d), jnp.float32),
        in_specs=[
            pl.BlockSpec(memory_space=pltpu.MemorySpace.HBM),
            pl.BlockSpec(memory_space=pltpu.MemorySpace.HBM),
        ],
        out_specs=pl.BlockSpec(memory_space=pltpu.VMEM),
        scratch_shapes=[
            pltpu.VMEM((m, slab), jnp.float32),
            pltpu.VMEM((m, slab), jnp.float32),
            pltpu.VMEM((m, d), jnp.float32),
            pltpu.VMEM((15 * CHUNK, d), jnp.bfloat16),
            pltpu.VMEM((15 * CHUNK, d), jnp.bfloat16),
            pltpu.VMEM((m, d), jnp.bfloat16),
            pltpu.SemaphoreType.DMA((2,)),
            pltpu.SemaphoreType.DMA((2 * N_STEPS, NS)),
            pltpu.SemaphoreType.DMA((2 * N_STEPS, NS)),
        ],
        compiler_params=pltpu.CompilerParams(
            vmem_limit_bytes=100 * 1024 * 1024,
        ),
    )(dy, W)


# device time: 57768 ns/iter; 4.5549x vs baseline; 1.0995x over previous
import jax
import jax.numpy as jnp
from jax import lax
from jax.experimental import pallas as pl
from jax.experimental.pallas import tpu as pltpu

N_SLABS = 8
NS = 4
CHUNK = 64
RS_OFF = (0, 8, 12, 14)
N_STEPS = 4


def kernel(dy, W):
    m, k_shard = dy.shape
    d = W.shape[0]
    slab = k_shard // N_SLABS
    csz = d // NS

    def body(dy_hbm, w_hbm, out_ref, a_buf, b_buf, acc, sbuf, rbuf, agbuf,
             in_sems, send_sems, recv_sems):
        x = lax.axis_index("x")
        y = lax.axis_index("y")
        z = lax.axis_index("z")
        c = x * 4 + z

        cpa = pltpu.make_async_copy(
            dy_hbm.at[:, pl.ds(c * slab, slab)], a_buf, in_sems.at[0])
        cpb = pltpu.make_async_copy(
            w_hbm.at[:, pl.ds(c * slab, slab)], b_buf, in_sems.at[1])
        cpa.start()
        cpb.start()
        cpa.wait()
        cpb.wait()
        acc[...] = lax.dot_general(
            a_buf[...].astype(jnp.bfloat16),
            b_buf[...].astype(jnp.bfloat16),
            dimension_numbers=(((1,), (1,)), ((), ())),
            preferred_element_type=jnp.float32,
        )

        info = {
            8: ((1 - x, y, z), x),
            4: ((x, 1 - y, z), y),
            1: ((x, y, jnp.bitwise_xor(z, 1)), z & 1),
            2: ((x, y, jnp.bitwise_xor(z, 2)), z >> 1),
        }
        base = (8, 4, 1, 2)
        orders = [
            tuple(base[(s + j * (N_STEPS // NS)) % N_STEPS] for s in range(N_STEPS))
            for j in range(NS)
        ]

        lo = [jnp.int32(0)] * NS
        sz = 16
        for s in range(N_STEPS):
            half = sz // 2
            rows = half * CHUNK
            off = RS_OFF[s] * CHUNK
            started = []
            for j in range(NS):
                pid, mybit = info[orders[j][s]]
                keep_lo = lo[j] + mybit * half
                send_lo = lo[j] + (1 - mybit) * half
                cols = pl.ds(j * csz, csz)
                sbuf[pl.ds(off, rows), cols] = (
                    acc[pl.ds(send_lo * CHUNK, rows), cols].astype(jnp.bfloat16))
                rdma = pltpu.make_async_remote_copy(
                    src_ref=sbuf.at[pl.ds(off, rows), cols],
                    dst_ref=rbuf.at[pl.ds(off, rows), cols],
                    send_sem=send_sems.at[s, j],
                    recv_sem=recv_sems.at[s, j],
                    device_id=pid,
                    device_id_type=pl.DeviceIdType.MESH,
                )
                rdma.start()
                started.append((rdma, keep_lo))
            for j, (rdma, keep_lo) in enumerate(started):
                cols = pl.ds(j * csz, csz)
                rdma.wait()
                acc[pl.ds(keep_lo * CHUNK, rows), cols] += (
                    rbuf[pl.ds(off, rows), cols].astype(jnp.float32))
                lo[j] = keep_lo
            sz = half

        for j in range(NS):
            cols = pl.ds(j * csz, csz)
            agbuf[pl.ds(lo[j] * CHUNK, CHUNK), cols] = (
                acc[pl.ds(lo[j] * CHUNK, CHUNK), cols].astype(jnp.bfloat16))

        sz = 1
        for s in range(N_STEPS):
            rows = sz * CHUNK
            started = []
            for j in range(NS):
                pid, mybit = info[orders[j][N_STEPS - 1 - s]]
                cols = pl.ds(j * csz, csz)
                rdma = pltpu.make_async_remote_copy(
                    src_ref=agbuf.at[pl.ds(lo[j] * CHUNK, rows), cols],
                    dst_ref=agbuf.at[pl.ds(lo[j] * CHUNK, rows), cols],
                    send_sem=send_sems.at[N_STEPS + s, j],
                    recv_sem=recv_sems.at[N_STEPS + s, j],
                    device_id=pid,
                    device_id_type=pl.DeviceIdType.MESH,
                )
                rdma.start()
                started.append((rdma, mybit))
            for j, (rdma, mybit) in enumerate(started):
                rdma.wait()
                lo[j] = lo[j] - mybit * sz
            sz *= 2

        out_ref[...] = agbuf[...].astype(jnp.float32)

    return pl.pallas_call(
        body,
        out_shape=jax.ShapeDtypeStruct((m, d), jnp.float32),
        in_specs=[
            pl.BlockSpec(memory_space=pltpu.MemorySpace.HBM),
            pl.BlockSpec(memory_space=pltpu.MemorySpace.HBM),
        ],
        out_specs=pl.BlockSpec(memory_space=pltpu.VMEM),
        scratch_shapes=[
            pltpu.VMEM((m, slab), jnp.float32),
            pltpu.VMEM((m, slab), jnp.float32),
            pltpu.VMEM((m, d), jnp.float32),
            pltpu.VMEM((15 * CHUNK, d), jnp.bfloat16),
            pltpu.VMEM((15 * CHUNK, d), jnp.bfloat16),
            pltpu.VMEM((m, d), jnp.bfloat16),
            pltpu.SemaphoreType.DMA((2,)),
            pltpu.SemaphoreType.DMA((2 * N_STEPS, NS)),
            pltpu.SemaphoreType.DMA((2 * N_STEPS, NS)),
        ],
        compiler_params=pltpu.CompilerParams(
            vmem_limit_bytes=100 * 1024 * 1024,
        ),
    )(dy, W)


# device time: 45538 ns/iter; 5.7781x vs baseline; 1.2686x over previous
import jax
import jax.numpy as jnp
from jax import lax
from jax.experimental import pallas as pl
from jax.experimental.pallas import tpu as pltpu

N_SLABS = 8
NS = 4
CHUNK = 64
RS_OFF = (0, 8, 12, 14)
N_STEPS = 4


def kernel(dy, W):
    m, k_shard = dy.shape
    d = W.shape[0]
    slab = k_shard // N_SLABS
    csz = d // NS

    def body(dy_hbm, w_hbm, out_ref, a_buf, b_buf, acc, rbuf, agbuf,
             in_sems, send_sems, recv_sems):
        x = lax.axis_index("x")
        y = lax.axis_index("y")
        z = lax.axis_index("z")
        c = x * 4 + z

        cpa = pltpu.make_async_copy(
            dy_hbm.at[:, pl.ds(c * slab, slab)], a_buf, in_sems.at[0])
        cpb = pltpu.make_async_copy(
            w_hbm.at[:, pl.ds(c * slab, slab)], b_buf, in_sems.at[1])
        cpa.start()
        cpb.start()

        neighbors = [
            (1 - x, y, z),
            (x, 1 - y, z),
            (x, y, jnp.bitwise_xor(z, 1)),
            (x, y, jnp.bitwise_xor(z, 2)),
        ]
        bsem = pltpu.get_barrier_semaphore()
        for pid in neighbors:
            pl.semaphore_signal(bsem, inc=1, device_id=pid,
                                device_id_type=pl.DeviceIdType.MESH)
        pl.semaphore_wait(bsem, 4)

        cpa.wait()
        cpb.wait()
        acc[...] = lax.dot_general(
            a_buf[...].astype(jnp.bfloat16),
            b_buf[...].astype(jnp.bfloat16),
            dimension_numbers=(((1,), (1,)), ((), ())),
            preferred_element_type=jnp.float32,
        ).astype(jnp.bfloat16)

        info = {
            8: (neighbors[0], x),
            4: (neighbors[1], y),
            1: (neighbors[2], z & 1),
            2: (neighbors[3], z >> 1),
        }
        base = (8, 4, 1, 2)
        orders = [
            tuple(base[(s + j * (N_STEPS // NS)) % N_STEPS] for s in range(N_STEPS))
            for j in range(NS)
        ]
        col = [pl.ds(j * csz, csz) for j in range(NS)]
        drain = []

        def start_rs(s, j, lo_j):
            half = 8 >> s
            rows = half * CHUNK
            pid, mybit = info[orders[j][s]]
            keep_lo = lo_j + mybit * half
            send_lo = lo_j + (1 - mybit) * half
            rdma = pltpu.make_async_remote_copy(
                src_ref=acc.at[pl.ds(send_lo * CHUNK, rows), col[j]],
                dst_ref=rbuf.at[pl.ds(RS_OFF[s] * CHUNK, rows), col[j]],
                send_sem=send_sems.at[s, j],
                recv_sem=recv_sems.at[s, j],
                device_id=pid,
                device_id_type=pl.DeviceIdType.MESH,
            )
            rdma.start()
            return rdma, keep_lo

        def start_ag(s, j, lo_j):
            rows = (1 << s) * CHUNK
            pid, mybit = info[orders[j][N_STEPS - 1 - s]]
            rdma = pltpu.make_async_remote_copy(
                src_ref=agbuf.at[pl.ds(lo_j * CHUNK, rows), col[j]],
                dst_ref=agbuf.at[pl.ds(lo_j * CHUNK, rows), col[j]],
                send_sem=send_sems.at[N_STEPS + s, j],
                recv_sem=recv_sems.at[N_STEPS + s, j],
                device_id=pid,
                device_id_type=pl.DeviceIdType.MESH,
            )
            rdma.start()
            return rdma, mybit

        lo = [jnp.int32(0)] * NS
        R, keep = [None] * NS, [None] * NS
        for j in range(NS):
            R[j], keep[j] = start_rs(0, j, lo[j])
        abit = [None] * NS
        for s in range(N_STEPS):
            rows = (8 >> s) * CHUNK
            off = RS_OFF[s] * CHUNK
            for j in range(NS):
                R[j].wait_recv()
                drain.append(R[j])
                acc[pl.ds(keep[j] * CHUNK, rows), col[j]] += (
                    rbuf[pl.ds(off, rows), col[j]])
                lo[j] = keep[j]
                if s < N_STEPS - 1:
                    R[j], keep[j] = start_rs(s + 1, j, lo[j])
                else:
                    agbuf[pl.ds(lo[j] * CHUNK, CHUNK), col[j]] = (
                        acc[pl.ds(lo[j] * CHUNK, CHUNK), col[j]])
                    R[j], abit[j] = start_ag(0, j, lo[j])

        for s in range(N_STEPS):
            for j in range(NS):
                R[j].wait_recv()
                drain.append(R[j])
                lo[j] = lo[j] - abit[j] * (1 << s)
                if s < N_STEPS - 1:
                    R[j], abit[j] = start_ag(s + 1, j, lo[j])
                else:
                    out_ref[:, col[j]] = agbuf[:, col[j]].astype(jnp.float32)

        for rdma in drain:
            rdma.wait_send()

    return pl.pallas_call(
        body,
        out_shape=jax.ShapeDtypeStruct((m, d), jnp.float32),
        in_specs=[
            pl.BlockSpec(memory_space=pltpu.MemorySpace.HBM),
            pl.BlockSpec(memory_space=pltpu.MemorySpace.HBM),
        ],
        out_specs=pl.BlockSpec(memory_space=pltpu.VMEM),
        scratch_shapes=[
            pltpu.VMEM((m, slab), jnp.float32),
            pltpu.VMEM((m, slab), jnp.float32),
            pltpu.VMEM((m, d), jnp.bfloat16),
            pltpu.VMEM((15 * CHUNK, d), jnp.bfloat16),
            pltpu.VMEM((m, d), jnp.bfloat16),
            pltpu.SemaphoreType.DMA((2,)),
            pltpu.SemaphoreType.DMA((2 * N_STEPS, NS)),
            pltpu.SemaphoreType.DMA((2 * N_STEPS, NS)),
        ],
        compiler_params=pltpu.CompilerParams(
            collective_id=0,
            vmem_limit_bytes=100 * 1024 * 1024,
        ),
    )(dy, W)
